# baseline (speedup 1.0000x reference)
import functools

import jax
import jax.numpy as jnp
from jax import lax
from jax.experimental import pallas as pl
from jax.experimental.pallas import tpu as pltpu
from jax.experimental.pallas import tpu_sc as plsc

_NC = 2
_NS = 16
_NW = _NC * _NS

_WIDE = 128
_LANES = 16


def _transform_table(tableT, W, b):
    D, V = tableT.shape
    blk = 12800
    grid = pl.cdiv(V, blk)

    def body(t_ref, w_ref, b_ref, o_ref):
        o_ref[:, :D] = (
            lax.dot_general(
                t_ref[...],
                w_ref[...],
                dimension_numbers=(((0,), (0,)), ((), ())),
                preferred_element_type=jnp.float32,
            )
            + b_ref[...][None, :]
        )

    return pl.pallas_call(
        body,
        grid=(grid,),
        in_specs=[
            pl.BlockSpec((D, blk), lambda i: (0, i)),
            pl.BlockSpec((D, D), lambda i: (0, 0)),
            pl.BlockSpec((D,), lambda i: (0,)),
        ],
        out_specs=pl.BlockSpec((blk, _WIDE), lambda i: (i, 0)),
        out_shape=jax.ShapeDtypeStruct((V, _WIDE), jnp.float32),
    )(tableT, W, b)


def _gather_transpose(L, D, Bsz):
    BB = Bsz // _NW
    HB = BB // 2
    assert BB == 128 and L % 2 == 0 and D == 64
    mesh = plsc.VectorSubcoreMesh(core_axis_name="c", subcore_axis_name="s")

    @functools.partial(
        pl.kernel,
        out_type=jax.ShapeDtypeStruct((L, D, Bsz), jnp.float32),
        mesh=mesh,
        scratch_types=[
            pltpu.VMEM((L, BB), jnp.int32),
            pltpu.VMEM((2, 2, HB, _WIDE), jnp.float32),
            pltpu.VMEM((2, D, BB), jnp.float32),
            pltpu.SemaphoreType.DMA,
            pltpu.SemaphoreType.DMA,
        ],
        compiler_params=pltpu.CompilerParams(needs_layout_passes=False),
    )
    def gather_k(tok_hbm, t2_hbm, out_hbm, idx_v, rows_v, trans_v, gsem, ssem):
        wid = lax.axis_index("s") * _NC + lax.axis_index("c")
        b0 = wid * BB
        pltpu.sync_copy(tok_hbm.at[:, pl.ds(b0, BB)], idx_v)
        lanes = lax.iota(jnp.int32, _LANES)

        def fire_g(l, p):
            for h in range(2):
                pltpu.async_copy(
                    t2_hbm.at[idx_v.at[l, pl.ds(h * HB, HB)]],
                    rows_v.at[p, h],
                    gsem,
                )

        def drain_g(l, p):
            for h in range(2):
                pltpu.make_async_copy(
                    t2_hbm.at[idx_v.at[l, pl.ds(h * HB, HB)]],
                    rows_v.at[p, h],
                    gsem,
                ).wait()

        def fire_s(l, p):
            pltpu.async_copy(
                trans_v.at[p], out_hbm.at[l, :, pl.ds(b0, BB)], ssem
            )

        def drain_s(l, p):
            pltpu.make_async_copy(
                trans_v.at[p], out_hbm.at[l, :, pl.ds(b0, BB)], ssem
            ).wait()

        def transpose(p):
            def dbody(d, carry):
                col = jnp.full((_LANES,), d, jnp.int32)
                for h in range(2):
                    for kg in range(HB // _LANES):
                        v = plsc.load_gather(
                            rows_v.at[p, h], [lanes + (kg * _LANES), col]
                        )
                        trans_v[p, d, pl.ds(h * HB + kg * _LANES, _LANES)] = v
                return carry

            lax.fori_loop(0, D, dbody, 0)

        fire_g(0, 0)
        drain_g(0, 0)
        fire_g(1, 1)
        transpose(0)
        fire_s(0, 0)

        def body(k, carry):
            l = 2 * k + 1
            drain_s(l - 1, 0)
            drain_g(l, 1)
            fire_g(l + 1, 0)
            transpose(1)
            fire_s(l, 1)
            l2 = 2 * k + 2
            drain_s(l2 - 1, 1)
            drain_g(l2, 0)
            fire_g(l2 + 1, 1)
            transpose(0)
            fire_s(l2, 0)
            return carry

        lax.fori_loop(0, L // 2 - 1, body, 0)

        drain_s(L - 2, 0)
        drain_g(L - 1, 1)
        transpose(1)
        fire_s(L - 1, 1)
        drain_s(L - 1, 1)

    return gather_k


def kernel(token_ids, table, W, b):
    Bsz, L = token_ids.shape
    V, D = table.shape
    assert Bsz % _NW == 0

    table2 = _transform_table(table.T, W, b)
    tokT = token_ids.T
    outP = _gather_transpose(L, D, Bsz)(tokT, table2)
    return jnp.transpose(outP, (2, 0, 1))


# speedup vs baseline: 1.3223x; 1.3223x over previous
import functools

import jax
import jax.numpy as jnp
from jax import lax
from jax.experimental import pallas as pl
from jax.experimental.pallas import tpu as pltpu
from jax.experimental.pallas import tpu_sc as plsc

_NC = 2
_NS = 16
_NW = _NC * _NS

_CH = 128


def _transform_table(tableT, W, b):
    D, V = tableT.shape
    blk = 12800
    grid = pl.cdiv(V, blk)

    def body(t_ref, w_ref, b_ref, o_ref):
        o_ref[...] = (
            lax.dot_general(
                t_ref[...],
                w_ref[...],
                dimension_numbers=(((0,), (0,)), ((), ())),
                preferred_element_type=jnp.float32,
            )
            + b_ref[...][None, :]
        )

    return pl.pallas_call(
        body,
        grid=(grid,),
        in_specs=[
            pl.BlockSpec((D, blk), lambda i: (0, i)),
            pl.BlockSpec((D, D), lambda i: (0, 0)),
            pl.BlockSpec((D,), lambda i: (0,)),
        ],
        out_specs=pl.BlockSpec((blk, D), lambda i: (i, 0)),
        out_shape=jax.ShapeDtypeStruct((V, D), jnp.float32),
    )(tableT, W, b)


def _gather(N, D):
    per = N // _NW
    NCH = per // _CH
    assert per % _CH == 0 and NCH % 2 == 0
    mesh = plsc.VectorSubcoreMesh(core_axis_name="c", subcore_axis_name="s")

    @functools.partial(
        pl.kernel,
        out_type=jax.ShapeDtypeStruct((N, D), jnp.float32),
        mesh=mesh,
        scratch_types=[
            pltpu.VMEM((NCH, _CH), jnp.int32),
            pltpu.VMEM((2, _CH, D), jnp.float32),
            pltpu.SemaphoreType.DMA,
            pltpu.SemaphoreType.DMA,
        ],
        compiler_params=pltpu.CompilerParams(use_tc_tiling_on_sc=False),
    )
    def gather_k(tok_hbm, t2_hbm, out_hbm, idx_v, rows_v, gsem, ssem):
        wid = lax.axis_index("s") * _NC + lax.axis_index("c")
        base = wid * per
        pltpu.sync_copy(tok_hbm.at[pl.ds(wid * NCH, NCH)], idx_v)

        def fire_g(c, p):
            pltpu.async_copy(t2_hbm.at[idx_v.at[c]], rows_v.at[p], gsem)

        def drain_g(c, p):
            pltpu.make_async_copy(
                t2_hbm.at[idx_v.at[c]], rows_v.at[p], gsem
            ).wait()

        def fire_s(c, p):
            pltpu.async_copy(
                rows_v.at[p], out_hbm.at[pl.ds(base + c * _CH, _CH)], ssem
            )

        def drain_s(c, p):
            pltpu.make_async_copy(
                rows_v.at[p], out_hbm.at[pl.ds(base + c * _CH, _CH)], ssem
            ).wait()

        fire_g(0, 0)
        drain_g(0, 0)
        fire_s(0, 0)
        fire_g(1, 1)

        def body(k, carry):
            c1 = 2 * k + 1
            drain_g(c1, 1)
            fire_s(c1, 1)
            drain_s(c1 - 1, 0)
            fire_g(c1 + 1, 0)
            c2 = 2 * k + 2
            drain_g(c2, 0)
            fire_s(c2, 0)
            drain_s(c2 - 1, 1)
            fire_g(c2 + 1, 1)
            return carry

        lax.fori_loop(0, NCH // 2 - 1, body, 0)

        drain_g(NCH - 1, 1)
        fire_s(NCH - 1, 1)
        drain_s(NCH - 2, 0)
        drain_s(NCH - 1, 1)

    return gather_k


def kernel(token_ids, table, W, b):
    Bsz, L = token_ids.shape
    V, D = table.shape
    N = Bsz * L
    assert N % (_NW * _CH) == 0

    table2 = _transform_table(table.T, W, b)
    tok2d = token_ids.reshape(N // _CH, _CH)
    out = _gather(N, D)(tok2d, table2)
    return out.reshape(Bsz, L, D)
